# Initial kernel scaffold; baseline (speedup 1.0000x reference)
import jax
import jax.numpy as jnp
from jax.experimental import pallas as pl


def kernel(t, W):
    raise NotImplementedError("write your pallas kernel here")


# speedup vs baseline: 15.3849x; 15.3849x over previous
import functools

import jax
import jax.numpy as jnp
from jax import lax
from jax.experimental import pallas as pl
from jax.experimental.pallas import tpu as pltpu
from jax.experimental.pallas import tpu_sc as plsc

NUM_ROWS = 100000
EMB = 32
KH = 4
HC = 73244475
HC0 = HC & 0xFFFF
HC1 = HC >> 16
M32 = 16824
M64C = 100000 - 51616

N_TOK = 16384 * 26
LANES = 128
ROWS2 = N_TOK // LANES
HB = 128
NW = 32
TOK_PER_W = N_TOK // NW
BLK = 512
NBLK = TOK_PER_W // BLK
IDX_ROWS = KH * BLK // LANES


def _hash_block_i64(t_ref, o_ref):
    x = t_ref[...].astype(jnp.uint32)
    outs = []
    for r in range(KH):
        lo = x + jnp.uint32(r)
        hi = jnp.zeros_like(lo)
        for _ in range(2):
            s_lo = ((lo >> 16) | (hi << 16)) ^ lo
            s_hi = (hi.astype(jnp.int32) >> 16).astype(jnp.uint32) ^ hi
            m_lo = s_lo * jnp.uint32(HC)
            x0 = s_lo & jnp.uint32(0xFFFF)
            x1 = s_lo >> 16
            m0 = x0 * jnp.uint32(HC0)
            mid = x1 * jnp.uint32(HC0) + x0 * jnp.uint32(HC1) + (m0 >> 16)
            mhi = x1 * jnp.uint32(HC1) + (mid >> 16)
            hi = s_hi * jnp.uint32(HC) + mhi
            lo = m_lo
        f_lo = ((lo >> 16) | (hi << 16)) ^ lo
        f_hi = (hi.astype(jnp.int32) >> 16).astype(jnp.uint32) ^ hi
        m = jnp.uint32(NUM_ROWS)
        p = ((f_hi % m) * jnp.uint32(M32)) % m
        q = (jnp.uint32(4) * p + f_lo % m) % m
        q = jnp.where(f_hi.astype(jnp.int32) < 0, (q + jnp.uint32(M64C)) % m, q)
        outs.append(q.astype(jnp.int32))
    o_ref[...] = jnp.stack(outs)


def _hash_block_i32(t_ref, o_ref):
    outs = []
    for r in range(KH):
        t = t_ref[...] + r
        t = ((t >> 16) ^ t) * HC
        t = ((t >> 16) ^ t) * HC
        t = (t >> 16) ^ t
        outs.append(jnp.mod(t, NUM_ROWS))
    o_ref[...] = jnp.stack(outs)


def _compute_idx(t32, exact_i64):
    body = _hash_block_i64 if exact_i64 else _hash_block_i32
    return pl.pallas_call(
        body,
        grid=(ROWS2 // HB,),
        in_specs=[pl.BlockSpec((HB, LANES), lambda i: (i, jnp.int32(0)))],
        out_specs=pl.BlockSpec(
            (KH, HB, LANES), lambda i: (jnp.int32(0), i, jnp.int32(0))
        ),
        out_shape=jax.ShapeDtypeStruct((KH, ROWS2, LANES), jnp.int32),
    )(t32)


def _gather_mean(idx, w):
    mesh = plsc.VectorSubcoreMesh(core_axis_name="c", subcore_axis_name="s")

    @functools.partial(
        pl.kernel,
        out_type=jax.ShapeDtypeStruct((N_TOK, EMB), jnp.float32),
        mesh=mesh,
        compiler_params=pltpu.CompilerParams(use_tc_tiling_on_sc=False),
        scratch_types=[
            pltpu.VMEM((KH * BLK,), jnp.int32),
            pltpu.VMEM((KH * BLK, EMB), jnp.float32),
            pltpu.VMEM((BLK, EMB), jnp.float32),
            pltpu.SemaphoreType.DMA,
        ],
    )
    def k(idx_hbm, w_hbm, out_hbm, idx_v, rows_v, out_v, sem):
        wid = lax.axis_index("s") * 2 + lax.axis_index("c")

        def blk_body(b, carry):
            tok0 = wid * jnp.int32(TOK_PER_W) + b * jnp.int32(BLK)
            for r in range(KH):
                pltpu.sync_copy(
                    idx_hbm.at[pl.ds(jnp.int32(r * N_TOK) + tok0, BLK)],
                    idx_v.at[pl.ds(r * BLK, BLK)],
                )
            copies = [
                pltpu.async_copy(
                    w_hbm.at[idx_v.at[pl.ds(c * LANES, LANES)]],
                    rows_v.at[pl.ds(c * LANES, LANES)],
                    sem,
                )
                for c in range(IDX_ROWS)
            ]
            for cp in copies:
                cp.wait()

            def red(i, c2):
                for j in range(EMB // 16):
                    sl = pl.ds(j * 16, 16)
                    acc = (
                        rows_v[i, sl]
                        + rows_v[BLK + i, sl]
                        + rows_v[2 * BLK + i, sl]
                        + rows_v[3 * BLK + i, sl]
                    )
                    out_v[i, sl] = acc * (1.0 / KH)
                return c2

            lax.fori_loop(jnp.int32(0), jnp.int32(BLK), red, 0)
            pltpu.sync_copy(out_v, out_hbm.at[pl.ds(tok0, BLK)])
            return carry

        lax.fori_loop(jnp.int32(0), jnp.int32(NBLK), blk_body, 0)

    return k(idx, w)


def kernel(t, W):
    exact_i64 = t.dtype == jnp.int64
    t32 = t.reshape(ROWS2, LANES).astype(jnp.int32)
    idx = _compute_idx(t32, exact_i64).reshape(-1)
    out = _gather_mean(idx, W)
    return out.reshape(t.shape + (EMB,))
